# Initial kernel scaffold; baseline (speedup 1.0000x reference)
import functools

import jax
import jax.numpy as jnp
from jax import lax
from jax.experimental import pallas as pl
from jax.experimental.pallas import tpu as pltpu
from jax.experimental.pallas import tpu_sc as plsc

N = 10001
D = 128
E = 320000
NPAD = 10240
NC = 2
NS = 16
NW = NC * NS
EPW = E // NW
C = 80
NCHUNK = EPW // C
L = 16
SLAB = NPAD // NS
ROWS_W = NPAD // NW


def _mesh():
    return plsc.VectorSubcoreMesh(core_axis_name="c", subcore_axis_name="s")


def _rsqrt16(d):
    bits = plsc.bitcast(d, jnp.int32)
    i = jnp.int32(0x5F3759DF) - lax.shift_right_logical(bits, 1)
    y = plsc.bitcast(i, jnp.float32)
    for _ in range(3):
        y = y * (1.5 - 0.5 * d * y * y)
    return jnp.where(d > 0.5, y, 0.0)


def _propagate(row_v, nrm_v, col_v, ridx, cidx, rows, x_hbm, out_sh, sem):
    def chunk(k, carry):
        o = k * C
        pltpu.sync_copy(row_v.at[pl.ds(o, C)], ridx)
        pltpu.sync_copy(col_v.at[pl.ds(o, C)], cidx)
        pltpu.async_copy(x_hbm.at[ridx], rows, sem).wait()

        def scale(e, carry2):
            nv = nrm_v[o + e]
            for j in range(D // L):
                rows[e, pl.ds(j * L, L)] = rows[e, pl.ds(j * L, L)] * nv
            return carry2

        lax.fori_loop(0, C, scale, 0)
        pltpu.sync_copy(rows, out_sh.at[cidx], add=True)
        return carry

    lax.fori_loop(0, NCHUNK, chunk, 0)


@functools.partial(
    pl.kernel,
    out_type=jax.ShapeDtypeStruct((NC, NPAD), jnp.float32),
    mesh=_mesh(),
    scratch_types=[
        pltpu.VMEM((EPW,), jnp.int32),
        pltpu.VMEM((C,), jnp.int32),
        pltpu.VMEM((C,), jnp.float32),
        pltpu.VMEM_SHARED((NPAD,), jnp.float32),
    ],
)
def _deg_kernel(edge_hbm, zeros_hbm, degp_hbm, col_v, cidx, ones_v, deg_sh):
    c = lax.axis_index("c")
    s = lax.axis_index("s")
    wid = s * NC + c

    @pl.when(s == 0)
    def _zero():
        pltpu.sync_copy(zeros_hbm, deg_sh)

    pltpu.sync_copy(edge_hbm.at[1, pl.ds(wid * EPW, EPW)], col_v)

    def fill(i, carry):
        ones_v[pl.ds(i * L, L)] = jnp.full((L,), 1.0, jnp.float32)
        return carry

    lax.fori_loop(0, C // L, fill, 0)
    plsc.subcore_barrier()

    def chunk(k, carry):
        pltpu.sync_copy(col_v.at[pl.ds(k * C, C)], cidx)
        pltpu.sync_copy(ones_v, deg_sh.at[cidx], add=True)
        return carry

    lax.fori_loop(0, NCHUNK, chunk, 0)
    plsc.subcore_barrier()

    @pl.when(s == 0)
    def _out():
        pltpu.sync_copy(deg_sh, degp_hbm.at[c])


@functools.partial(
    pl.kernel,
    out_type=(
        jax.ShapeDtypeStruct((NC, NPAD, D), jnp.float32),
        jax.ShapeDtypeStruct((E,), jnp.float32),
    ),
    mesh=_mesh(),
    scratch_types=[
        pltpu.VMEM((EPW,), jnp.int32),
        pltpu.VMEM((EPW,), jnp.int32),
        pltpu.VMEM((EPW,), jnp.float32),
        pltpu.VMEM((NPAD,), jnp.float32),
        pltpu.VMEM((NPAD,), jnp.float32),
        pltpu.VMEM((C,), jnp.int32),
        pltpu.VMEM((C,), jnp.int32),
        pltpu.VMEM((C, D), jnp.float32),
        pltpu.VMEM_SHARED((NPAD, D), jnp.float32),
        pltpu.SemaphoreType.DMA,
    ],
)
def _layer1_kernel(edge_hbm, ew_hbm, x0_hbm, degp_hbm, zrow_hbm,
                   part_hbm, norm_hbm,
                   row_v, col_v, nrm_v, dis_v, tmp_v, ridx, cidx, rows,
                   out_sh, sem):
    c = lax.axis_index("c")
    s = lax.axis_index("s")
    wid = s * NC + c
    base = wid * EPW

    pltpu.sync_copy(zrow_hbm, out_sh.at[pl.ds(s * SLAB, SLAB)])

    pltpu.sync_copy(degp_hbm.at[0], dis_v)
    pltpu.sync_copy(degp_hbm.at[1], tmp_v)

    def mkdis(i, carry):
        dv = dis_v[pl.ds(i * L, L)] + tmp_v[pl.ds(i * L, L)]
        dis_v[pl.ds(i * L, L)] = _rsqrt16(dv)
        return carry

    lax.fori_loop(0, NPAD // L, mkdis, 0)

    pltpu.sync_copy(edge_hbm.at[0, pl.ds(base, EPW)], row_v)
    pltpu.sync_copy(edge_hbm.at[1, pl.ds(base, EPW)], col_v)
    pltpu.sync_copy(ew_hbm.at[pl.ds(base, EPW)], nrm_v)

    def mknorm(i, carry):
        r = row_v[pl.ds(i * L, L)]
        cc = col_v[pl.ds(i * L, L)]
        dr = plsc.load_gather(dis_v, (r,))
        dc = plsc.load_gather(dis_v, (cc,))
        nrm_v[pl.ds(i * L, L)] = nrm_v[pl.ds(i * L, L)] * dr * dc
        return carry

    lax.fori_loop(0, EPW // L, mknorm, 0)
    pltpu.sync_copy(nrm_v, norm_hbm.at[pl.ds(base, EPW)])

    plsc.subcore_barrier()
    _propagate(row_v, nrm_v, col_v, ridx, cidx, rows, x0_hbm, out_sh, sem)
    plsc.subcore_barrier()
    pltpu.sync_copy(out_sh.at[pl.ds(s * SLAB, SLAB)],
                    part_hbm.at[c, pl.ds(s * SLAB, SLAB)])


@functools.partial(
    pl.kernel,
    out_type=jax.ShapeDtypeStruct((NC, NPAD, D), jnp.float32),
    mesh=_mesh(),
    scratch_types=[
        pltpu.VMEM((EPW,), jnp.int32),
        pltpu.VMEM((EPW,), jnp.int32),
        pltpu.VMEM((EPW,), jnp.float32),
        pltpu.VMEM((C,), jnp.int32),
        pltpu.VMEM((C,), jnp.int32),
        pltpu.VMEM((C, D), jnp.float32),
        pltpu.VMEM_SHARED((NPAD, D), jnp.float32),
        pltpu.SemaphoreType.DMA,
    ],
)
def _layer2_kernel(edge_hbm, norm_hbm, x_hbm, zrow_hbm, part_hbm,
                   row_v, col_v, nrm_v, ridx, cidx, rows, out_sh, sem):
    c = lax.axis_index("c")
    s = lax.axis_index("s")
    wid = s * NC + c
    base = wid * EPW

    pltpu.sync_copy(zrow_hbm, out_sh.at[pl.ds(s * SLAB, SLAB)])
    pltpu.sync_copy(edge_hbm.at[0, pl.ds(base, EPW)], row_v)
    pltpu.sync_copy(edge_hbm.at[1, pl.ds(base, EPW)], col_v)
    pltpu.sync_copy(norm_hbm.at[pl.ds(base, EPW)], nrm_v)

    plsc.subcore_barrier()
    _propagate(row_v, nrm_v, col_v, ridx, cidx, rows, x_hbm, out_sh, sem)
    plsc.subcore_barrier()
    pltpu.sync_copy(out_sh.at[pl.ds(s * SLAB, SLAB)],
                    part_hbm.at[c, pl.ds(s * SLAB, SLAB)])


def _vadd_rows(a_v, b_v, scale=None):
    def add(i, carry):
        for j in range(D // L):
            v = a_v[i, pl.ds(j * L, L)] + b_v[i, pl.ds(j * L, L)]
            if scale is not None:
                v = v * scale
            a_v[i, pl.ds(j * L, L)] = v
        return carry

    lax.fori_loop(0, ROWS_W, add, 0)


@functools.partial(
    pl.kernel,
    out_type=jax.ShapeDtypeStruct((NPAD, D), jnp.float32),
    mesh=_mesh(),
    scratch_types=[
        pltpu.VMEM((ROWS_W, D), jnp.float32),
        pltpu.VMEM((ROWS_W, D), jnp.float32),
    ],
)
def _combine_kernel(part_hbm, x_hbm, a_v, b_v):
    c = lax.axis_index("c")
    s = lax.axis_index("s")
    wid = s * NC + c
    r0 = wid * ROWS_W
    pltpu.sync_copy(part_hbm.at[0, pl.ds(r0, ROWS_W)], a_v)
    pltpu.sync_copy(part_hbm.at[1, pl.ds(r0, ROWS_W)], b_v)
    _vadd_rows(a_v, b_v)
    pltpu.sync_copy(a_v, x_hbm.at[pl.ds(r0, ROWS_W)])


@functools.partial(
    pl.kernel,
    out_type=jax.ShapeDtypeStruct((NPAD, D), jnp.float32),
    mesh=_mesh(),
    scratch_types=[
        pltpu.VMEM((ROWS_W, D), jnp.float32),
        pltpu.VMEM((ROWS_W, D), jnp.float32),
    ],
)
def _final_kernel(part_hbm, x1_hbm, x0_hbm, out_hbm, a_v, b_v):
    c = lax.axis_index("c")
    s = lax.axis_index("s")
    wid = s * NC + c
    r0 = wid * ROWS_W
    pltpu.sync_copy(part_hbm.at[0, pl.ds(r0, ROWS_W)], a_v)
    pltpu.sync_copy(part_hbm.at[1, pl.ds(r0, ROWS_W)], b_v)
    _vadd_rows(a_v, b_v)
    pltpu.sync_copy(x1_hbm.at[pl.ds(r0, ROWS_W)], b_v)
    _vadd_rows(a_v, b_v)
    pltpu.sync_copy(x0_hbm.at[pl.ds(r0, ROWS_W)], b_v)
    _vadd_rows(a_v, b_v, scale=jnp.float32(1.0 / 3.0))
    pltpu.sync_copy(a_v, out_hbm.at[pl.ds(r0, ROWS_W)])


def kernel(edge_index, edge_weight, item_emb):
    x0 = jnp.zeros((NPAD, D), jnp.float32).at[:N].set(item_emb)
    zeros_deg = jnp.zeros((NPAD,), jnp.float32)
    zrow = jnp.zeros((SLAB, D), jnp.float32)
    degp = _deg_kernel(edge_index, zeros_deg)
    part1, norm = _layer1_kernel(edge_index, edge_weight, x0, degp, zrow)
    x1 = _combine_kernel(part1)
    part2 = _layer2_kernel(edge_index, norm, x1, zrow)
    final = _final_kernel(part2, x1, x0)
    return final[:N]


# speedup vs baseline: 11.6341x; 11.6341x over previous
import functools

import jax
import jax.numpy as jnp
from jax import lax
from jax.experimental import pallas as pl
from jax.experimental.pallas import tpu as pltpu
from jax.experimental.pallas import tpu_sc as plsc

N = 10001
D = 128
E = 320000
NPAD = 10240
NC = 2
NS = 16
NW = NC * NS
EPW = E // NW
C = 80
NCHUNK = EPW // C
BLK = 2000
NBLK = EPW // BLK
L = 16
SLAB = NPAD // NS
ROWS_W = NPAD // NW


def _mesh():
    return plsc.VectorSubcoreMesh(core_axis_name="c", subcore_axis_name="s")


def _rsqrt16(d):
    bits = lax.bitcast_convert_type(d, jnp.int32)
    i = jnp.int32(0x5F3759DF) - lax.shift_right_logical(bits, 1)
    y = lax.bitcast_convert_type(i, jnp.float32)
    for _ in range(3):
        y = y * (1.5 - 0.5 * d * y * y)
    return jnp.where(d > 0.5, y, 0.0)


def _propagate_block(row_b, nrm_b, col_b, ridx, cidx, rows, x_hbm, out_sh,
                     sem):
    def chunk(k, carry):
        o = k * C
        for g in range(C // L):
            ridx[pl.ds(g * L, L)] = row_b[pl.ds(o + g * L, L)]
            cidx[pl.ds(g * L, L)] = col_b[pl.ds(o + g * L, L)]
        pltpu.async_copy(x_hbm.at[ridx], rows, sem).wait()

        def scale(g, carry2):
            nv16 = nrm_b[pl.ds(o + g * L, L)]
            for e in range(L):
                nv = nv16[e]
                ri = g * L + e
                for j in range(D // L):
                    rows[ri, pl.ds(j * L, L)] = rows[ri, pl.ds(j * L, L)] * nv
            return carry2

        lax.fori_loop(0, C // L, scale, 0)
        pltpu.sync_copy(rows, out_sh.at[cidx], add=True)
        return carry

    lax.fori_loop(0, BLK // C, chunk, 0)


@functools.partial(
    pl.kernel,
    out_type=jax.ShapeDtypeStruct((NC, NPAD), jnp.float32),
    mesh=_mesh(),
    compiler_params=pltpu.CompilerParams(needs_layout_passes=False),
    scratch_types=[
        pltpu.VMEM((EPW,), jnp.int32),
        pltpu.VMEM((C,), jnp.int32),
        pltpu.VMEM((C,), jnp.float32),
        pltpu.VMEM_SHARED((NPAD,), jnp.float32),
    ],
)
def _deg_kernel(colall_hbm, zeros_hbm, degp_hbm, col_v, cidx, ones_v, deg_sh):
    c = lax.axis_index("c")
    s = lax.axis_index("s")
    wid = s * NC + c

    @pl.when(s == 0)
    def _zero():
        pltpu.sync_copy(zeros_hbm, deg_sh)

    pltpu.sync_copy(colall_hbm.at[pl.ds(wid * EPW, EPW)], col_v)

    def fill(i, carry):
        ones_v[pl.ds(i * L, L)] = jnp.full((L,), 1.0, jnp.float32)
        return carry

    lax.fori_loop(0, C // L, fill, 0)
    plsc.subcore_barrier()

    def chunk(k, carry):
        for g in range(C // L):
            cidx[pl.ds(g * L, L)] = col_v[pl.ds(k * C + g * L, L)]
        pltpu.sync_copy(ones_v, deg_sh.at[cidx], add=True)
        return carry

    lax.fori_loop(0, NCHUNK, chunk, 0)
    plsc.subcore_barrier()

    @pl.when(s == 0)
    def _out():
        pltpu.sync_copy(deg_sh, degp_hbm.at[c])


@functools.partial(
    pl.kernel,
    out_type=(
        jax.ShapeDtypeStruct((NC, NPAD, D), jnp.float32),
        jax.ShapeDtypeStruct((E,), jnp.float32),
    ),
    mesh=_mesh(),
    compiler_params=pltpu.CompilerParams(needs_layout_passes=False),
    scratch_types=[
        pltpu.VMEM((BLK,), jnp.int32),
        pltpu.VMEM((BLK,), jnp.int32),
        pltpu.VMEM((BLK,), jnp.float32),
        pltpu.VMEM((NPAD,), jnp.float32),
        pltpu.VMEM((NPAD,), jnp.float32),
        pltpu.VMEM((C,), jnp.int32),
        pltpu.VMEM((C,), jnp.int32),
        pltpu.VMEM((C, D), jnp.float32),
        pltpu.VMEM_SHARED((NPAD, D), jnp.float32),
        pltpu.SemaphoreType.DMA,
    ],
)
def _layer1_kernel(rowall_hbm, colall_hbm, ew_hbm, x0_hbm, degp_hbm, zrow_hbm,
                   part_hbm, norm_hbm,
                   row_b, col_b, nrm_b, dis_v, tmp_v, ridx, cidx, rows,
                   out_sh, sem):
    c = lax.axis_index("c")
    s = lax.axis_index("s")
    wid = s * NC + c
    base = wid * EPW

    pltpu.sync_copy(zrow_hbm, out_sh.at[pl.ds(s * SLAB, SLAB)])

    pltpu.sync_copy(degp_hbm.at[0], dis_v)
    pltpu.sync_copy(degp_hbm.at[1], tmp_v)

    def mkdis(i, carry):
        dv = dis_v[pl.ds(i * L, L)] + tmp_v[pl.ds(i * L, L)]
        dis_v[pl.ds(i * L, L)] = _rsqrt16(dv)
        return carry

    lax.fori_loop(0, NPAD // L, mkdis, 0)
    plsc.subcore_barrier()

    def block(b, carry):
        bo = base + b * BLK
        pltpu.sync_copy(rowall_hbm.at[pl.ds(bo, BLK)], row_b)
        pltpu.sync_copy(colall_hbm.at[pl.ds(bo, BLK)], col_b)
        pltpu.sync_copy(ew_hbm.at[pl.ds(bo, BLK)], nrm_b)

        def mknorm(i, carry2):
            r = row_b[pl.ds(i * L, L)]
            cc = col_b[pl.ds(i * L, L)]
            dr = plsc.load_gather(dis_v, (r,))
            dc = plsc.load_gather(dis_v, (cc,))
            nrm_b[pl.ds(i * L, L)] = nrm_b[pl.ds(i * L, L)] * dr * dc
            return carry2

        lax.fori_loop(0, BLK // L, mknorm, 0)
        pltpu.sync_copy(nrm_b, norm_hbm.at[pl.ds(bo, BLK)])
        _propagate_block(row_b, nrm_b, col_b, ridx, cidx, rows, x0_hbm,
                         out_sh, sem)
        return carry

    lax.fori_loop(0, NBLK, block, 0)
    plsc.subcore_barrier()
    pltpu.sync_copy(out_sh.at[pl.ds(s * SLAB, SLAB)],
                    part_hbm.at[c, pl.ds(s * SLAB, SLAB)])


@functools.partial(
    pl.kernel,
    out_type=jax.ShapeDtypeStruct((NC, NPAD, D), jnp.float32),
    mesh=_mesh(),
    compiler_params=pltpu.CompilerParams(needs_layout_passes=False),
    scratch_types=[
        pltpu.VMEM((BLK,), jnp.int32),
        pltpu.VMEM((BLK,), jnp.int32),
        pltpu.VMEM((BLK,), jnp.float32),
        pltpu.VMEM((C,), jnp.int32),
        pltpu.VMEM((C,), jnp.int32),
        pltpu.VMEM((C, D), jnp.float32),
        pltpu.VMEM_SHARED((NPAD, D), jnp.float32),
        pltpu.SemaphoreType.DMA,
    ],
)
def _layer2_kernel(rowall_hbm, colall_hbm, norm_hbm, x_hbm, zrow_hbm, part_hbm,
                   row_b, col_b, nrm_b, ridx, cidx, rows, out_sh, sem):
    c = lax.axis_index("c")
    s = lax.axis_index("s")
    wid = s * NC + c
    base = wid * EPW

    pltpu.sync_copy(zrow_hbm, out_sh.at[pl.ds(s * SLAB, SLAB)])
    plsc.subcore_barrier()

    def block(b, carry):
        bo = base + b * BLK
        pltpu.sync_copy(rowall_hbm.at[pl.ds(bo, BLK)], row_b)
        pltpu.sync_copy(colall_hbm.at[pl.ds(bo, BLK)], col_b)
        pltpu.sync_copy(norm_hbm.at[pl.ds(bo, BLK)], nrm_b)
        _propagate_block(row_b, nrm_b, col_b, ridx, cidx, rows, x_hbm,
                         out_sh, sem)
        return carry

    lax.fori_loop(0, NBLK, block, 0)
    plsc.subcore_barrier()
    pltpu.sync_copy(out_sh.at[pl.ds(s * SLAB, SLAB)],
                    part_hbm.at[c, pl.ds(s * SLAB, SLAB)])


def _vadd_rows(a_v, b_v, scale=None):
    def add(i, carry):
        for j in range(D // L):
            v = a_v[i, pl.ds(j * L, L)] + b_v[i, pl.ds(j * L, L)]
            if scale is not None:
                v = v * scale
            a_v[i, pl.ds(j * L, L)] = v
        return carry

    lax.fori_loop(0, ROWS_W, add, 0)


@functools.partial(
    pl.kernel,
    out_type=jax.ShapeDtypeStruct((NPAD, D), jnp.float32),
    mesh=_mesh(),
    compiler_params=pltpu.CompilerParams(needs_layout_passes=False),
    scratch_types=[
        pltpu.VMEM((ROWS_W, D), jnp.float32),
        pltpu.VMEM((ROWS_W, D), jnp.float32),
    ],
)
def _combine_kernel(part_hbm, x_hbm, a_v, b_v):
    c = lax.axis_index("c")
    s = lax.axis_index("s")
    wid = s * NC + c
    r0 = wid * ROWS_W
    pltpu.sync_copy(part_hbm.at[0, pl.ds(r0, ROWS_W)], a_v)
    pltpu.sync_copy(part_hbm.at[1, pl.ds(r0, ROWS_W)], b_v)
    _vadd_rows(a_v, b_v)
    pltpu.sync_copy(a_v, x_hbm.at[pl.ds(r0, ROWS_W)])


@functools.partial(
    pl.kernel,
    out_type=jax.ShapeDtypeStruct((NPAD, D), jnp.float32),
    mesh=_mesh(),
    compiler_params=pltpu.CompilerParams(needs_layout_passes=False),
    scratch_types=[
        pltpu.VMEM((ROWS_W, D), jnp.float32),
        pltpu.VMEM((ROWS_W, D), jnp.float32),
    ],
)
def _final_kernel(part_hbm, x1_hbm, x0_hbm, out_hbm, a_v, b_v):
    c = lax.axis_index("c")
    s = lax.axis_index("s")
    wid = s * NC + c
    r0 = wid * ROWS_W
    pltpu.sync_copy(part_hbm.at[0, pl.ds(r0, ROWS_W)], a_v)
    pltpu.sync_copy(part_hbm.at[1, pl.ds(r0, ROWS_W)], b_v)
    _vadd_rows(a_v, b_v)
    pltpu.sync_copy(x1_hbm.at[pl.ds(r0, ROWS_W)], b_v)
    _vadd_rows(a_v, b_v)
    pltpu.sync_copy(x0_hbm.at[pl.ds(r0, ROWS_W)], b_v)
    _vadd_rows(a_v, b_v, scale=jnp.float32(1.0 / 3.0))
    pltpu.sync_copy(a_v, out_hbm.at[pl.ds(r0, ROWS_W)])


def kernel(edge_index, edge_weight, item_emb):
    x0 = jnp.zeros((NPAD, D), jnp.float32).at[:N].set(item_emb)
    zeros_deg = jnp.zeros((NPAD,), jnp.float32)
    zrow = jnp.zeros((SLAB, D), jnp.float32)
    rows_a = edge_index[0]
    cols_a = edge_index[1]
    degp = _deg_kernel(cols_a, zeros_deg)
    part1, norm = _layer1_kernel(rows_a, cols_a, edge_weight, x0, degp, zrow)
    x1 = _combine_kernel(part1)
    part2 = _layer2_kernel(rows_a, cols_a, norm, x1, zrow)
    final = _final_kernel(part2, x1, x0)
    return final[:N]


# speedup vs baseline: 17.2166x; 1.4798x over previous
import functools

import jax
import jax.numpy as jnp
from jax import lax
from jax.experimental import pallas as pl
from jax.experimental.pallas import tpu as pltpu
from jax.experimental.pallas import tpu_sc as plsc

N = 10001
D = 128
E = 320000
NPAD = 10240
NC = 2
NS = 16
NW = NC * NS
EPW = E // NW
C = 80
NCHUNK = EPW // C
BLK = 2000
NBLK = EPW // BLK
PIECE = 2048
L = 16
SLAB = NPAD // NS
ROWS_W = NPAD // NW


def _mesh():
    return plsc.VectorSubcoreMesh(core_axis_name="c", subcore_axis_name="s")


def _rsqrt16(d):
    bits = lax.bitcast_convert_type(d, jnp.int32)
    i = jnp.int32(0x5F3759DF) - lax.shift_right_logical(bits, 1)
    y = lax.bitcast_convert_type(i, jnp.float32)
    for _ in range(3):
        y = y * (1.5 - 0.5 * d * y * y)
    return jnp.where(d > 0.5, y, 0.0)


def _propagate_pipe(base, rowall, colall, nrmall, x_hbm, out_sh,
                    row_b, col_b, nrm_b, rows2, ridx2, cidx2, nidx2, gsem2):
    NCHW = EPW // C
    NCH_BLK = BLK // C

    def stage(k):
        @pl.when(lax.rem(k, NCH_BLK) == 0)
        def _():
            bo = base + lax.div(k, NCH_BLK) * BLK
            pltpu.sync_copy(rowall.at[pl.ds(bo, BLK)], row_b)
            pltpu.sync_copy(colall.at[pl.ds(bo, BLK)], col_b)
            pltpu.sync_copy(nrmall.at[pl.ds(bo, BLK)], nrm_b)

    def ib_g(k, p):
        ob = lax.rem(k * C, BLK)
        for g in range(C // L):
            ridx2[p][pl.ds(g * L, L)] = row_b[pl.ds(ob + g * L, L)]
            cidx2[p][pl.ds(g * L, L)] = col_b[pl.ds(ob + g * L, L)]
            nidx2[p][pl.ds(g * L, L)] = nrm_b[pl.ds(ob + g * L, L)]
        pltpu.async_copy(x_hbm.at[ridx2[p]], rows2[p], gsem2[p])

    def finish(k, p):
        pltpu.make_async_copy(x_hbm.at[ridx2[p]], rows2[p], gsem2[p]).wait()

        def scale(g, carry):
            nv16 = nidx2[p][pl.ds(g * L, L)]
            for e in range(L):
                nv = nv16[e]
                ri = g * L + e
                for j in range(D // L):
                    rows2[p][ri, pl.ds(j * L, L)] = (
                        rows2[p][ri, pl.ds(j * L, L)] * nv)
            return carry

        lax.fori_loop(0, C // L, scale, 0)
        pltpu.sync_copy(rows2[p], out_sh.at[cidx2[p]], add=True)

    stage(jnp.int32(0))
    ib_g(jnp.int32(0), 0)

    def pair(t, carry):
        ka = 2 * t + 1
        stage(ka)
        ib_g(ka, 1)
        finish(ka - 1, 0)
        kb = 2 * t + 2
        stage(kb)
        ib_g(kb, 0)
        finish(kb - 1, 1)
        return carry

    lax.fori_loop(0, (NCHW - 3) // 2, pair, 0)
    ib_g(jnp.int32(NCHW - 2), 1)
    finish(jnp.int32(NCHW - 3), 0)
    ib_g(jnp.int32(NCHW - 1), 0)
    finish(jnp.int32(NCHW - 2), 1)
    finish(jnp.int32(NCHW - 1), 0)


@functools.partial(
    pl.kernel,
    out_type=jax.ShapeDtypeStruct((NC, NPAD), jnp.float32),
    mesh=_mesh(),
    compiler_params=pltpu.CompilerParams(needs_layout_passes=False),
    scratch_types=[
        pltpu.VMEM((EPW,), jnp.int32),
        pltpu.VMEM((C,), jnp.int32),
        pltpu.VMEM((C,), jnp.float32),
        pltpu.VMEM_SHARED((NPAD,), jnp.float32),
    ],
)
def _deg_kernel(colall_hbm, zeros_hbm, degp_hbm, col_v, cidx, ones_v, deg_sh):
    c = lax.axis_index("c")
    s = lax.axis_index("s")
    wid = s * NC + c

    @pl.when(s == 0)
    def _zero():
        pltpu.sync_copy(zeros_hbm, deg_sh)

    pltpu.sync_copy(colall_hbm.at[pl.ds(wid * EPW, EPW)], col_v)

    def fill(i, carry):
        ones_v[pl.ds(i * L, L)] = jnp.full((L,), 1.0, jnp.float32)
        return carry

    lax.fori_loop(0, C // L, fill, 0)
    plsc.subcore_barrier()

    def chunk(k, carry):
        for g in range(C // L):
            cidx[pl.ds(g * L, L)] = col_v[pl.ds(k * C + g * L, L)]
        pltpu.sync_copy(ones_v, deg_sh.at[cidx], add=True)
        return carry

    lax.fori_loop(0, NCHUNK, chunk, 0)
    plsc.subcore_barrier()

    @pl.when(s == 0)
    def _out():
        pltpu.sync_copy(deg_sh, degp_hbm.at[c])


@functools.partial(
    pl.kernel,
    out_type=(
        jax.ShapeDtypeStruct((NC, NPAD, D), jnp.float32),
        jax.ShapeDtypeStruct((E,), jnp.float32),
    ),
    mesh=_mesh(),
    compiler_params=pltpu.CompilerParams(needs_layout_passes=False),
    scratch_types=[
        pltpu.VMEM((BLK,), jnp.int32),
        pltpu.VMEM((BLK,), jnp.int32),
        pltpu.VMEM((BLK,), jnp.float32),
        pltpu.VMEM((NPAD,), jnp.float32),
        pltpu.VMEM((PIECE,), jnp.float32),
        [pltpu.VMEM((C, D), jnp.float32) for _ in range(2)],
        [pltpu.VMEM((C,), jnp.int32) for _ in range(2)],
        [pltpu.VMEM((C,), jnp.int32) for _ in range(2)],
        [pltpu.VMEM((C,), jnp.float32) for _ in range(2)],
        pltpu.VMEM_SHARED((NPAD, D), jnp.float32),
        [pltpu.SemaphoreType.DMA for _ in range(2)],
    ],
)
def _layer1_kernel(rowall_hbm, colall_hbm, ew_hbm, x0_hbm, degp_hbm, zrow_hbm,
                   part_hbm, norm_hbm,
                   row_b, col_b, nrm_b, dis_v, piece, rows2, ridx2, cidx2,
                   nidx2, out_sh, gsem2):
    c = lax.axis_index("c")
    s = lax.axis_index("s")
    wid = s * NC + c
    base = wid * EPW

    pltpu.sync_copy(zrow_hbm, out_sh.at[pl.ds(s * SLAB, SLAB)])

    pltpu.sync_copy(degp_hbm.at[0], dis_v)
    for pc in range(NPAD // PIECE):
        pltpu.sync_copy(degp_hbm.at[1, pl.ds(pc * PIECE, PIECE)], piece)

        def mkdis(i, carry, pc=pc):
            off = pc * PIECE + i * L
            dv = dis_v[pl.ds(off, L)] + piece[pl.ds(i * L, L)]
            dis_v[pl.ds(off, L)] = _rsqrt16(dv)
            return carry

        lax.fori_loop(0, PIECE // L, mkdis, 0)

    for b in range(NBLK):
        bo = base + b * BLK
        pltpu.sync_copy(rowall_hbm.at[pl.ds(bo, BLK)], row_b)
        pltpu.sync_copy(colall_hbm.at[pl.ds(bo, BLK)], col_b)
        pltpu.sync_copy(ew_hbm.at[pl.ds(bo, BLK)], nrm_b)

        def mknorm(i, carry):
            r = row_b[pl.ds(i * L, L)]
            cc = col_b[pl.ds(i * L, L)]
            dr = plsc.load_gather(dis_v, (r,))
            dc = plsc.load_gather(dis_v, (cc,))
            nrm_b[pl.ds(i * L, L)] = nrm_b[pl.ds(i * L, L)] * dr * dc
            return carry

        lax.fori_loop(0, BLK // L, mknorm, 0)
        pltpu.sync_copy(nrm_b, norm_hbm.at[pl.ds(bo, BLK)])

    plsc.subcore_barrier()
    _propagate_pipe(base, rowall_hbm, colall_hbm, norm_hbm, x0_hbm, out_sh,
                    row_b, col_b, nrm_b, rows2, ridx2, cidx2, nidx2, gsem2)
    plsc.subcore_barrier()
    pltpu.sync_copy(out_sh.at[pl.ds(s * SLAB, SLAB)],
                    part_hbm.at[c, pl.ds(s * SLAB, SLAB)])


@functools.partial(
    pl.kernel,
    out_type=jax.ShapeDtypeStruct((NC, NPAD, D), jnp.float32),
    mesh=_mesh(),
    compiler_params=pltpu.CompilerParams(needs_layout_passes=False),
    scratch_types=[
        pltpu.VMEM((BLK,), jnp.int32),
        pltpu.VMEM((BLK,), jnp.int32),
        pltpu.VMEM((BLK,), jnp.float32),
        [pltpu.VMEM((C, D), jnp.float32) for _ in range(2)],
        [pltpu.VMEM((C,), jnp.int32) for _ in range(2)],
        [pltpu.VMEM((C,), jnp.int32) for _ in range(2)],
        [pltpu.VMEM((C,), jnp.float32) for _ in range(2)],
        pltpu.VMEM_SHARED((NPAD, D), jnp.float32),
        [pltpu.SemaphoreType.DMA for _ in range(2)],
    ],
)
def _layer2_kernel(rowall_hbm, colall_hbm, norm_hbm, x_hbm, zrow_hbm, part_hbm,
                   row_b, col_b, nrm_b, rows2, ridx2, cidx2, nidx2, out_sh,
                   gsem2):
    c = lax.axis_index("c")
    s = lax.axis_index("s")
    wid = s * NC + c
    base = wid * EPW

    pltpu.sync_copy(zrow_hbm, out_sh.at[pl.ds(s * SLAB, SLAB)])
    plsc.subcore_barrier()
    _propagate_pipe(base, rowall_hbm, colall_hbm, norm_hbm, x_hbm, out_sh,
                    row_b, col_b, nrm_b, rows2, ridx2, cidx2, nidx2, gsem2)
    plsc.subcore_barrier()
    pltpu.sync_copy(out_sh.at[pl.ds(s * SLAB, SLAB)],
                    part_hbm.at[c, pl.ds(s * SLAB, SLAB)])


def _vadd_rows(a_v, b_v, scale=None):
    def add(i, carry):
        for j in range(D // L):
            v = a_v[i, pl.ds(j * L, L)] + b_v[i, pl.ds(j * L, L)]
            if scale is not None:
                v = v * scale
            a_v[i, pl.ds(j * L, L)] = v
        return carry

    lax.fori_loop(0, ROWS_W, add, 0)


@functools.partial(
    pl.kernel,
    out_type=jax.ShapeDtypeStruct((NPAD, D), jnp.float32),
    mesh=_mesh(),
    compiler_params=pltpu.CompilerParams(needs_layout_passes=False),
    scratch_types=[
        pltpu.VMEM((ROWS_W, D), jnp.float32),
        pltpu.VMEM((ROWS_W, D), jnp.float32),
    ],
)
def _combine_kernel(part_hbm, x_hbm, a_v, b_v):
    c = lax.axis_index("c")
    s = lax.axis_index("s")
    wid = s * NC + c
    r0 = wid * ROWS_W
    pltpu.sync_copy(part_hbm.at[0, pl.ds(r0, ROWS_W)], a_v)
    pltpu.sync_copy(part_hbm.at[1, pl.ds(r0, ROWS_W)], b_v)
    _vadd_rows(a_v, b_v)
    pltpu.sync_copy(a_v, x_hbm.at[pl.ds(r0, ROWS_W)])


@functools.partial(
    pl.kernel,
    out_type=jax.ShapeDtypeStruct((NPAD, D), jnp.float32),
    mesh=_mesh(),
    compiler_params=pltpu.CompilerParams(needs_layout_passes=False),
    scratch_types=[
        pltpu.VMEM((ROWS_W, D), jnp.float32),
        pltpu.VMEM((ROWS_W, D), jnp.float32),
    ],
)
def _final_kernel(part_hbm, x1_hbm, x0_hbm, out_hbm, a_v, b_v):
    c = lax.axis_index("c")
    s = lax.axis_index("s")
    wid = s * NC + c
    r0 = wid * ROWS_W
    pltpu.sync_copy(part_hbm.at[0, pl.ds(r0, ROWS_W)], a_v)
    pltpu.sync_copy(part_hbm.at[1, pl.ds(r0, ROWS_W)], b_v)
    _vadd_rows(a_v, b_v)
    pltpu.sync_copy(x1_hbm.at[pl.ds(r0, ROWS_W)], b_v)
    _vadd_rows(a_v, b_v)
    pltpu.sync_copy(x0_hbm.at[pl.ds(r0, ROWS_W)], b_v)
    _vadd_rows(a_v, b_v, scale=jnp.float32(1.0 / 3.0))
    pltpu.sync_copy(a_v, out_hbm.at[pl.ds(r0, ROWS_W)])


def kernel(edge_index, edge_weight, item_emb):
    x0 = jnp.zeros((NPAD, D), jnp.float32).at[:N].set(item_emb)
    zeros_deg = jnp.zeros((NPAD,), jnp.float32)
    zrow = jnp.zeros((SLAB, D), jnp.float32)
    rows_a = edge_index[0]
    cols_a = edge_index[1]
    degp = _deg_kernel(cols_a, zeros_deg)
    part1, norm = _layer1_kernel(rows_a, cols_a, edge_weight, x0, degp, zrow)
    x1 = _combine_kernel(part1)
    part2 = _layer2_kernel(rows_a, cols_a, norm, x1, zrow)
    final = _final_kernel(part2, x1, x0)
    return final[:N]


# speedup vs baseline: 17.3537x; 1.0080x over previous
import functools

import jax
import jax.numpy as jnp
from jax import lax
from jax.experimental import pallas as pl
from jax.experimental.pallas import tpu as pltpu
from jax.experimental.pallas import tpu_sc as plsc

N = 10001
D = 128
E = 320000
NPAD = 10240
NC = 2
NS = 16
NW = NC * NS
EPW = E // NW
C = 80
NCHUNK = EPW // C
BLK = 2000
NBLK = EPW // BLK
PIECE = 2048
L = 16
SLAB = NPAD // NS
ROWS_W = NPAD // NW


def _mesh():
    return plsc.VectorSubcoreMesh(core_axis_name="c", subcore_axis_name="s")


def _rsqrt16(d):
    bits = lax.bitcast_convert_type(d, jnp.int32)
    i = jnp.int32(0x5F3759DF) - lax.shift_right_logical(bits, 1)
    y = lax.bitcast_convert_type(i, jnp.float32)
    for _ in range(3):
        y = y * (1.5 - 0.5 * d * y * y)
    return jnp.where(d > 0.5, y, 0.0)


def _propagate_pipe(base, rowall, colall, nrmall, x_hbm, out_sh,
                    row_b, col_b, nrm_b, rows2, ridx2, cidx2, nidx2, gsem2,
                    ssem2):
    NCHW = EPW // C
    NCH_BLK = BLK // C

    def stage(k):
        @pl.when(lax.rem(k, NCH_BLK) == 0)
        def _():
            bo = base + lax.div(k, NCH_BLK) * BLK
            pltpu.sync_copy(rowall.at[pl.ds(bo, BLK)], row_b)
            pltpu.sync_copy(colall.at[pl.ds(bo, BLK)], col_b)
            pltpu.sync_copy(nrmall.at[pl.ds(bo, BLK)], nrm_b)

    def ws(p):
        pltpu.make_async_copy(rows2[p], out_sh.at[cidx2[p]], ssem2[p]).wait()

    def ib_g(k, p):
        ob = lax.rem(k * C, BLK)
        for g in range(C // L):
            ridx2[p][pl.ds(g * L, L)] = row_b[pl.ds(ob + g * L, L)]
            cidx2[p][pl.ds(g * L, L)] = col_b[pl.ds(ob + g * L, L)]
            nidx2[p][pl.ds(g * L, L)] = nrm_b[pl.ds(ob + g * L, L)]
        pltpu.async_copy(x_hbm.at[ridx2[p]], rows2[p], gsem2[p])

    def finish(k, p):
        pltpu.make_async_copy(x_hbm.at[ridx2[p]], rows2[p], gsem2[p]).wait()

        def scale(g, carry):
            nv16 = nidx2[p][pl.ds(g * L, L)]
            for e in range(L):
                nv = nv16[e]
                ri = g * L + e
                for j in range(D // L):
                    rows2[p][ri, pl.ds(j * L, L)] = (
                        rows2[p][ri, pl.ds(j * L, L)] * nv)
            return carry

        lax.fori_loop(0, C // L, scale, 0)
        pltpu.async_copy(rows2[p], out_sh.at[cidx2[p]], ssem2[p], add=True)

    stage(jnp.int32(0))
    ib_g(jnp.int32(0), 0)

    def pair(t, carry):
        ka = 2 * t + 1
        stage(ka)

        @pl.when(t > 0)
        def _():
            ws(1)

        ib_g(ka, 1)
        finish(ka - 1, 0)
        kb = 2 * t + 2
        stage(kb)
        ws(0)
        ib_g(kb, 0)
        finish(kb - 1, 1)
        return carry

    lax.fori_loop(0, (NCHW - 3) // 2, pair, 0)
    ws(1)
    ib_g(jnp.int32(NCHW - 2), 1)
    finish(jnp.int32(NCHW - 3), 0)
    ws(0)
    ib_g(jnp.int32(NCHW - 1), 0)
    finish(jnp.int32(NCHW - 2), 1)
    ws(1)
    finish(jnp.int32(NCHW - 1), 0)
    ws(0)


@functools.partial(
    pl.kernel,
    out_type=jax.ShapeDtypeStruct((NC, NPAD), jnp.float32),
    mesh=_mesh(),
    compiler_params=pltpu.CompilerParams(needs_layout_passes=False),
    scratch_types=[
        pltpu.VMEM((EPW,), jnp.int32),
        pltpu.VMEM((C,), jnp.int32),
        pltpu.VMEM((C,), jnp.float32),
        pltpu.VMEM_SHARED((NPAD,), jnp.float32),
    ],
)
def _deg_kernel(colall_hbm, zeros_hbm, degp_hbm, col_v, cidx, ones_v, deg_sh):
    c = lax.axis_index("c")
    s = lax.axis_index("s")
    wid = s * NC + c

    @pl.when(s == 0)
    def _zero():
        pltpu.sync_copy(zeros_hbm, deg_sh)

    pltpu.sync_copy(colall_hbm.at[pl.ds(wid * EPW, EPW)], col_v)

    def fill(i, carry):
        ones_v[pl.ds(i * L, L)] = jnp.full((L,), 1.0, jnp.float32)
        return carry

    lax.fori_loop(0, C // L, fill, 0)
    plsc.subcore_barrier()

    def chunk(k, carry):
        for g in range(C // L):
            cidx[pl.ds(g * L, L)] = col_v[pl.ds(k * C + g * L, L)]
        pltpu.sync_copy(ones_v, deg_sh.at[cidx], add=True)
        return carry

    lax.fori_loop(0, NCHUNK, chunk, 0)
    plsc.subcore_barrier()

    @pl.when(s == 0)
    def _out():
        pltpu.sync_copy(deg_sh, degp_hbm.at[c])


@functools.partial(
    pl.kernel,
    out_type=(
        jax.ShapeDtypeStruct((NC, NPAD, D), jnp.float32),
        jax.ShapeDtypeStruct((E,), jnp.float32),
    ),
    mesh=_mesh(),
    compiler_params=pltpu.CompilerParams(needs_layout_passes=False),
    scratch_types=[
        pltpu.VMEM((BLK,), jnp.int32),
        pltpu.VMEM((BLK,), jnp.int32),
        pltpu.VMEM((BLK,), jnp.float32),
        pltpu.VMEM((NPAD,), jnp.float32),
        pltpu.VMEM((PIECE,), jnp.float32),
        [pltpu.VMEM((C, D), jnp.float32) for _ in range(2)],
        [pltpu.VMEM((C,), jnp.int32) for _ in range(2)],
        [pltpu.VMEM((C,), jnp.int32) for _ in range(2)],
        [pltpu.VMEM((C,), jnp.float32) for _ in range(2)],
        pltpu.VMEM_SHARED((NPAD, D), jnp.float32),
        [pltpu.SemaphoreType.DMA for _ in range(2)],
        [pltpu.SemaphoreType.DMA for _ in range(2)],
    ],
)
def _layer1_kernel(rowall_hbm, colall_hbm, ew_hbm, x0_hbm, degp_hbm, zrow_hbm,
                   part_hbm, norm_hbm,
                   row_b, col_b, nrm_b, dis_v, piece, rows2, ridx2, cidx2,
                   nidx2, out_sh, gsem2, ssem2):
    c = lax.axis_index("c")
    s = lax.axis_index("s")
    wid = s * NC + c
    base = wid * EPW

    pltpu.sync_copy(zrow_hbm, out_sh.at[pl.ds(s * SLAB, SLAB)])

    pltpu.sync_copy(degp_hbm.at[0], dis_v)
    for pc in range(NPAD // PIECE):
        pltpu.sync_copy(degp_hbm.at[1, pl.ds(pc * PIECE, PIECE)], piece)

        def mkdis(i, carry, pc=pc):
            off = pc * PIECE + i * L
            dv = dis_v[pl.ds(off, L)] + piece[pl.ds(i * L, L)]
            dis_v[pl.ds(off, L)] = _rsqrt16(dv)
            return carry

        lax.fori_loop(0, PIECE // L, mkdis, 0)

    for b in range(NBLK):
        bo = base + b * BLK
        pltpu.sync_copy(rowall_hbm.at[pl.ds(bo, BLK)], row_b)
        pltpu.sync_copy(colall_hbm.at[pl.ds(bo, BLK)], col_b)
        pltpu.sync_copy(ew_hbm.at[pl.ds(bo, BLK)], nrm_b)

        def mknorm(i, carry):
            r = row_b[pl.ds(i * L, L)]
            cc = col_b[pl.ds(i * L, L)]
            dr = plsc.load_gather(dis_v, (r,))
            dc = plsc.load_gather(dis_v, (cc,))
            nrm_b[pl.ds(i * L, L)] = nrm_b[pl.ds(i * L, L)] * dr * dc
            return carry

        lax.fori_loop(0, BLK // L, mknorm, 0)
        pltpu.sync_copy(nrm_b, norm_hbm.at[pl.ds(bo, BLK)])

    plsc.subcore_barrier()
    _propagate_pipe(base, rowall_hbm, colall_hbm, norm_hbm, x0_hbm, out_sh,
                    row_b, col_b, nrm_b, rows2, ridx2, cidx2, nidx2, gsem2,
                    ssem2)
    plsc.subcore_barrier()
    pltpu.sync_copy(out_sh.at[pl.ds(s * SLAB, SLAB)],
                    part_hbm.at[c, pl.ds(s * SLAB, SLAB)])


@functools.partial(
    pl.kernel,
    out_type=jax.ShapeDtypeStruct((NC, NPAD, D), jnp.float32),
    mesh=_mesh(),
    compiler_params=pltpu.CompilerParams(needs_layout_passes=False),
    scratch_types=[
        pltpu.VMEM((BLK,), jnp.int32),
        pltpu.VMEM((BLK,), jnp.int32),
        pltpu.VMEM((BLK,), jnp.float32),
        [pltpu.VMEM((C, D), jnp.float32) for _ in range(2)],
        [pltpu.VMEM((C,), jnp.int32) for _ in range(2)],
        [pltpu.VMEM((C,), jnp.int32) for _ in range(2)],
        [pltpu.VMEM((C,), jnp.float32) for _ in range(2)],
        pltpu.VMEM_SHARED((NPAD, D), jnp.float32),
        [pltpu.SemaphoreType.DMA for _ in range(2)],
        [pltpu.SemaphoreType.DMA for _ in range(2)],
    ],
)
def _layer2_kernel(rowall_hbm, colall_hbm, norm_hbm, x_hbm, zrow_hbm, part_hbm,
                   row_b, col_b, nrm_b, rows2, ridx2, cidx2, nidx2, out_sh,
                   gsem2, ssem2):
    c = lax.axis_index("c")
    s = lax.axis_index("s")
    wid = s * NC + c
    base = wid * EPW

    pltpu.sync_copy(zrow_hbm, out_sh.at[pl.ds(s * SLAB, SLAB)])
    plsc.subcore_barrier()
    _propagate_pipe(base, rowall_hbm, colall_hbm, norm_hbm, x_hbm, out_sh,
                    row_b, col_b, nrm_b, rows2, ridx2, cidx2, nidx2, gsem2,
                    ssem2)
    plsc.subcore_barrier()
    pltpu.sync_copy(out_sh.at[pl.ds(s * SLAB, SLAB)],
                    part_hbm.at[c, pl.ds(s * SLAB, SLAB)])


def _vadd_rows(a_v, b_v, scale=None):
    def add(i, carry):
        for j in range(D // L):
            v = a_v[i, pl.ds(j * L, L)] + b_v[i, pl.ds(j * L, L)]
            if scale is not None:
                v = v * scale
            a_v[i, pl.ds(j * L, L)] = v
        return carry

    lax.fori_loop(0, ROWS_W, add, 0)


@functools.partial(
    pl.kernel,
    out_type=jax.ShapeDtypeStruct((NPAD, D), jnp.float32),
    mesh=_mesh(),
    compiler_params=pltpu.CompilerParams(needs_layout_passes=False),
    scratch_types=[
        pltpu.VMEM((ROWS_W, D), jnp.float32),
        pltpu.VMEM((ROWS_W, D), jnp.float32),
    ],
)
def _combine_kernel(part_hbm, x_hbm, a_v, b_v):
    c = lax.axis_index("c")
    s = lax.axis_index("s")
    wid = s * NC + c
    r0 = wid * ROWS_W
    pltpu.sync_copy(part_hbm.at[0, pl.ds(r0, ROWS_W)], a_v)
    pltpu.sync_copy(part_hbm.at[1, pl.ds(r0, ROWS_W)], b_v)
    _vadd_rows(a_v, b_v)
    pltpu.sync_copy(a_v, x_hbm.at[pl.ds(r0, ROWS_W)])


@functools.partial(
    pl.kernel,
    out_type=jax.ShapeDtypeStruct((NPAD, D), jnp.float32),
    mesh=_mesh(),
    compiler_params=pltpu.CompilerParams(needs_layout_passes=False),
    scratch_types=[
        pltpu.VMEM((ROWS_W, D), jnp.float32),
        pltpu.VMEM((ROWS_W, D), jnp.float32),
    ],
)
def _final_kernel(part_hbm, x1_hbm, x0_hbm, out_hbm, a_v, b_v):
    c = lax.axis_index("c")
    s = lax.axis_index("s")
    wid = s * NC + c
    r0 = wid * ROWS_W
    pltpu.sync_copy(part_hbm.at[0, pl.ds(r0, ROWS_W)], a_v)
    pltpu.sync_copy(part_hbm.at[1, pl.ds(r0, ROWS_W)], b_v)
    _vadd_rows(a_v, b_v)
    pltpu.sync_copy(x1_hbm.at[pl.ds(r0, ROWS_W)], b_v)
    _vadd_rows(a_v, b_v)
    pltpu.sync_copy(x0_hbm.at[pl.ds(r0, ROWS_W)], b_v)
    _vadd_rows(a_v, b_v, scale=jnp.float32(1.0 / 3.0))
    pltpu.sync_copy(a_v, out_hbm.at[pl.ds(r0, ROWS_W)])


def kernel(edge_index, edge_weight, item_emb):
    x0 = jnp.zeros((NPAD, D), jnp.float32).at[:N].set(item_emb)
    zeros_deg = jnp.zeros((NPAD,), jnp.float32)
    zrow = jnp.zeros((SLAB, D), jnp.float32)
    rows_a = edge_index[0]
    cols_a = edge_index[1]
    degp = _deg_kernel(cols_a, zeros_deg)
    part1, norm = _layer1_kernel(rows_a, cols_a, edge_weight, x0, degp, zrow)
    x1 = _combine_kernel(part1)
    part2 = _layer2_kernel(rows_a, cols_a, norm, x1, zrow)
    final = _final_kernel(part2, x1, x0)
    return final[:N]


# speedup vs baseline: 17.7905x; 1.0252x over previous
import functools

import jax
import jax.numpy as jnp
from jax import lax
from jax.experimental import pallas as pl
from jax.experimental.pallas import tpu as pltpu
from jax.experimental.pallas import tpu_sc as plsc

N = 10001
D = 128
E = 320000
NPAD = 10240
NC = 2
NS = 16
NW = NC * NS
EPW = E // NW
C = 80
NCHUNK = EPW // C
BLK = 2000
NBLK = EPW // BLK
PIECE = 2048
L = 16
SLAB = NPAD // NS
ROWS_W = NPAD // NW


def _mesh():
    return plsc.VectorSubcoreMesh(core_axis_name="c", subcore_axis_name="s")


def _rsqrt16(d):
    bits = lax.bitcast_convert_type(d, jnp.int32)
    i = jnp.int32(0x5F3759DF) - lax.shift_right_logical(bits, 1)
    y = lax.bitcast_convert_type(i, jnp.float32)
    for _ in range(3):
        y = y * (1.5 - 0.5 * d * y * y)
    return jnp.where(d > 0.5, y, 0.0)


NBUF = 3
NCHW = EPW // C


def _propagate_pipe(wid, packed_hbm, x_hbm, out_sh, dis_v,
                    rows3, pbuf3, scidx3, gsem3, isem3, ssem3):
    cbase = wid * NCHW

    def ild(k, p):
        pltpu.async_copy(packed_hbm.at[cbase + k], pbuf3[p], isem3[p])

    def wi_g(k, p):
        pltpu.make_async_copy(packed_hbm.at[cbase + k], pbuf3[p],
                              isem3[p]).wait()
        pltpu.async_copy(x_hbm.at[pbuf3[p].at[0]], rows3[p], gsem3[p])

    def ws(p):
        pltpu.make_async_copy(rows3[p], out_sh.at[scidx3[p]],
                              ssem3[p]).wait()

    def finish(k, p):
        pltpu.make_async_copy(x_hbm.at[pbuf3[p].at[0]], rows3[p],
                              gsem3[p]).wait()

        def scale(g, carry):
            r16 = pbuf3[p][0, pl.ds(g * L, L)]
            c16 = pbuf3[p][1, pl.ds(g * L, L)]
            w16 = lax.bitcast_convert_type(pbuf3[p][2, pl.ds(g * L, L)],
                                           jnp.float32)
            nv16 = (w16 * plsc.load_gather(dis_v, (r16,))
                    * plsc.load_gather(dis_v, (c16,)))
            scidx3[p][pl.ds(g * L, L)] = c16
            for e in range(L):
                nv = nv16[e]
                ri = g * L + e
                for j in range(D // L):
                    rows3[p][ri, pl.ds(j * L, L)] = (
                        rows3[p][ri, pl.ds(j * L, L)] * nv)
            return carry

        lax.fori_loop(0, C // L, scale, 0)
        pltpu.async_copy(rows3[p], out_sh.at[scidx3[p]], ssem3[p], add=True)

    z = jnp.int32(0)
    ild(z, 0)
    ild(z + 1, 1)
    ild(z + 2, 2)
    wi_g(z, 0)
    wi_g(z + 1, 1)
    wi_g(z + 2, 2)
    finish(z, 0)
    ild(z + 3, 0)

    def triple(t, carry):
        for d, p in ((3, 0), (4, 1), (5, 2)):
            k = 3 * t + d
            ws(p)
            wi_g(k, p)
            finish(k - 2, (p + 1) % 3)
            ild(k + 1, (p + 1) % 3)
        return carry

    lax.fori_loop(0, (NCHW - 5) // 3, triple, 0)
    k = jnp.int32(NCHW - 2)
    ws(0)
    wi_g(k, 0)
    finish(k - 2, 1)
    ild(k + 1, 1)
    ws(1)
    wi_g(k + 1, 1)
    finish(k - 1, 2)
    finish(k, 0)
    finish(k + 1, 1)
    ws(2)
    ws(0)
    ws(1)


@functools.partial(
    pl.kernel,
    out_type=jax.ShapeDtypeStruct((NC, NPAD), jnp.float32),
    mesh=_mesh(),
    compiler_params=pltpu.CompilerParams(needs_layout_passes=False),
    scratch_types=[
        pltpu.VMEM((EPW,), jnp.int32),
        pltpu.VMEM((C,), jnp.int32),
        pltpu.VMEM((C,), jnp.float32),
        pltpu.VMEM_SHARED((NPAD,), jnp.float32),
    ],
)
def _deg_kernel(colall_hbm, zeros_hbm, degp_hbm, col_v, cidx, ones_v, deg_sh):
    c = lax.axis_index("c")
    s = lax.axis_index("s")
    wid = s * NC + c

    @pl.when(s == 0)
    def _zero():
        pltpu.sync_copy(zeros_hbm, deg_sh)

    pltpu.sync_copy(colall_hbm.at[pl.ds(wid * EPW, EPW)], col_v)

    def fill(i, carry):
        ones_v[pl.ds(i * L, L)] = jnp.full((L,), 1.0, jnp.float32)
        return carry

    lax.fori_loop(0, C // L, fill, 0)
    plsc.subcore_barrier()

    def chunk(k, carry):
        for g in range(C // L):
            cidx[pl.ds(g * L, L)] = col_v[pl.ds(k * C + g * L, L)]
        pltpu.sync_copy(ones_v, deg_sh.at[cidx], add=True)
        return carry

    lax.fori_loop(0, NCHUNK, chunk, 0)
    plsc.subcore_barrier()

    @pl.when(s == 0)
    def _out():
        pltpu.sync_copy(deg_sh, degp_hbm.at[c])


@functools.partial(
    pl.kernel,
    out_type=jax.ShapeDtypeStruct((NC, NPAD, D), jnp.float32),
    mesh=_mesh(),
    compiler_params=pltpu.CompilerParams(needs_layout_passes=False),
    scratch_types=[
        pltpu.VMEM((NPAD,), jnp.float32),
        pltpu.VMEM((PIECE,), jnp.float32),
        [pltpu.VMEM((C, D), jnp.float32) for _ in range(NBUF)],
        [pltpu.VMEM((3, C), jnp.int32) for _ in range(NBUF)],
        [pltpu.VMEM((C,), jnp.int32) for _ in range(NBUF)],
        pltpu.VMEM_SHARED((NPAD, D), jnp.float32),
        [pltpu.SemaphoreType.DMA for _ in range(NBUF)],
        [pltpu.SemaphoreType.DMA for _ in range(NBUF)],
        [pltpu.SemaphoreType.DMA for _ in range(NBUF)],
    ],
)
def _layer_kernel(packed_hbm, x_hbm, degp_hbm, zrow_hbm, part_hbm,
                  dis_v, piece, rows3, pbuf3, scidx3,
                  out_sh, gsem3, isem3, ssem3):
    c = lax.axis_index("c")
    s = lax.axis_index("s")
    wid = s * NC + c

    pltpu.sync_copy(zrow_hbm, out_sh.at[pl.ds(s * SLAB, SLAB)])

    pltpu.sync_copy(degp_hbm.at[0], dis_v)
    for pc in range(NPAD // PIECE):
        pltpu.sync_copy(degp_hbm.at[1, pl.ds(pc * PIECE, PIECE)], piece)

        def mkdis(i, carry, pc=pc):
            off = pc * PIECE + i * L
            dv = dis_v[pl.ds(off, L)] + piece[pl.ds(i * L, L)]
            dis_v[pl.ds(off, L)] = _rsqrt16(dv)
            return carry

        lax.fori_loop(0, PIECE // L, mkdis, 0)

    plsc.subcore_barrier()
    _propagate_pipe(wid, packed_hbm, x_hbm, out_sh, dis_v,
                    rows3, pbuf3, scidx3, gsem3, isem3, ssem3)
    plsc.subcore_barrier()
    pltpu.sync_copy(out_sh.at[pl.ds(s * SLAB, SLAB)],
                    part_hbm.at[c, pl.ds(s * SLAB, SLAB)])


def _vadd_rows(a_v, b_v, scale=None):
    def add(i, carry):
        for j in range(D // L):
            v = a_v[i, pl.ds(j * L, L)] + b_v[i, pl.ds(j * L, L)]
            if scale is not None:
                v = v * scale
            a_v[i, pl.ds(j * L, L)] = v
        return carry

    lax.fori_loop(0, ROWS_W, add, 0)


@functools.partial(
    pl.kernel,
    out_type=jax.ShapeDtypeStruct((NPAD, D), jnp.float32),
    mesh=_mesh(),
    compiler_params=pltpu.CompilerParams(needs_layout_passes=False),
    scratch_types=[
        pltpu.VMEM((ROWS_W, D), jnp.float32),
        pltpu.VMEM((ROWS_W, D), jnp.float32),
    ],
)
def _combine_kernel(part_hbm, x_hbm, a_v, b_v):
    c = lax.axis_index("c")
    s = lax.axis_index("s")
    wid = s * NC + c
    r0 = wid * ROWS_W
    pltpu.sync_copy(part_hbm.at[0, pl.ds(r0, ROWS_W)], a_v)
    pltpu.sync_copy(part_hbm.at[1, pl.ds(r0, ROWS_W)], b_v)
    _vadd_rows(a_v, b_v)
    pltpu.sync_copy(a_v, x_hbm.at[pl.ds(r0, ROWS_W)])


@functools.partial(
    pl.kernel,
    out_type=jax.ShapeDtypeStruct((NPAD, D), jnp.float32),
    mesh=_mesh(),
    compiler_params=pltpu.CompilerParams(needs_layout_passes=False),
    scratch_types=[
        pltpu.VMEM((ROWS_W, D), jnp.float32),
        pltpu.VMEM((ROWS_W, D), jnp.float32),
    ],
)
def _final_kernel(part_hbm, x1_hbm, x0_hbm, out_hbm, a_v, b_v):
    c = lax.axis_index("c")
    s = lax.axis_index("s")
    wid = s * NC + c
    r0 = wid * ROWS_W
    pltpu.sync_copy(part_hbm.at[0, pl.ds(r0, ROWS_W)], a_v)
    pltpu.sync_copy(part_hbm.at[1, pl.ds(r0, ROWS_W)], b_v)
    _vadd_rows(a_v, b_v)
    pltpu.sync_copy(x1_hbm.at[pl.ds(r0, ROWS_W)], b_v)
    _vadd_rows(a_v, b_v)
    pltpu.sync_copy(x0_hbm.at[pl.ds(r0, ROWS_W)], b_v)
    _vadd_rows(a_v, b_v, scale=jnp.float32(1.0 / 3.0))
    pltpu.sync_copy(a_v, out_hbm.at[pl.ds(r0, ROWS_W)])


def kernel(edge_index, edge_weight, item_emb):
    x0 = jnp.zeros((NPAD, D), jnp.float32).at[:N].set(item_emb)
    zeros_deg = jnp.zeros((NPAD,), jnp.float32)
    zrow = jnp.zeros((SLAB, D), jnp.float32)
    rows_a = edge_index[0]
    cols_a = edge_index[1]
    ew_i = lax.bitcast_convert_type(edge_weight, jnp.int32)
    packed = jnp.stack([rows_a.reshape(E // C, C), cols_a.reshape(E // C, C),
                        ew_i.reshape(E // C, C)], axis=1)
    degp = _deg_kernel(cols_a, zeros_deg)
    part1 = _layer_kernel(packed, x0, degp, zrow)
    x1 = _combine_kernel(part1)
    part2 = _layer_kernel(packed, x1, degp, zrow)
    final = _final_kernel(part2, x1, x0)
    return final[:N]


# speedup vs baseline: 18.0468x; 1.0144x over previous
import functools

import jax
import jax.numpy as jnp
from jax import lax
from jax.experimental import pallas as pl
from jax.experimental.pallas import tpu as pltpu
from jax.experimental.pallas import tpu_sc as plsc

N = 10001
D = 128
E = 320000
NPAD = 10240
NC = 2
NS = 16
NW = NC * NS
EPW = E // NW
C = 80
NCHUNK = EPW // C
BLK = 2000
NBLK = EPW // BLK
PIECE = 2048
L = 16
SLAB = NPAD // NS
ROWS_W = NPAD // NW


def _mesh():
    return plsc.VectorSubcoreMesh(core_axis_name="c", subcore_axis_name="s")


def _rsqrt16(d):
    bits = lax.bitcast_convert_type(d, jnp.int32)
    i = jnp.int32(0x5F3759DF) - lax.shift_right_logical(bits, 1)
    y = lax.bitcast_convert_type(i, jnp.float32)
    for _ in range(3):
        y = y * (1.5 - 0.5 * d * y * y)
    return jnp.where(d > 0.5, y, 0.0)


NBUF = 3
NCHW = EPW // C


def _propagate_pipe(wid, packed_hbm, x_hbm, out_sh, dis_v,
                    rows3, pbuf3, scidx3, gsem3, isem3, ssem3):
    cbase = wid * NCHW

    def ild(k, p):
        pltpu.async_copy(packed_hbm.at[cbase + k], pbuf3[p], isem3[p])

    def wi_g(k, p):
        pltpu.make_async_copy(packed_hbm.at[cbase + k], pbuf3[p],
                              isem3[p]).wait()
        pltpu.async_copy(x_hbm.at[pbuf3[p].at[0]], rows3[p], gsem3[p])

    def ws(p):
        pltpu.make_async_copy(rows3[p], out_sh.at[scidx3[p]],
                              ssem3[p]).wait()

    def finish(k, p):
        pltpu.make_async_copy(x_hbm.at[pbuf3[p].at[0]], rows3[p],
                              gsem3[p]).wait()

        def scale(g, carry):
            r16 = pbuf3[p][0, pl.ds(g * L, L)]
            c16 = pbuf3[p][1, pl.ds(g * L, L)]
            w16 = lax.bitcast_convert_type(pbuf3[p][2, pl.ds(g * L, L)],
                                           jnp.float32)
            nv16 = (w16 * plsc.load_gather(dis_v, (r16,))
                    * plsc.load_gather(dis_v, (c16,)))
            scidx3[p][pl.ds(g * L, L)] = c16
            for e in range(L):
                nv = nv16[e]
                ri = g * L + e
                for j in range(D // L):
                    rows3[p][ri, pl.ds(j * L, L)] = (
                        rows3[p][ri, pl.ds(j * L, L)] * nv)
            return carry

        lax.fori_loop(0, C // L, scale, 0)
        pltpu.async_copy(rows3[p], out_sh.at[scidx3[p]], ssem3[p], add=True)

    z = jnp.int32(0)
    ild(z, 0)
    ild(z + 1, 1)
    ild(z + 2, 2)
    wi_g(z, 0)
    wi_g(z + 1, 1)
    wi_g(z + 2, 2)
    finish(z, 0)
    ild(z + 3, 0)

    def triple(t, carry):
        for d, p in ((3, 0), (4, 1), (5, 2)):
            k = 3 * t + d
            ws(p)
            wi_g(k, p)
            finish(k - 2, (p + 1) % 3)
            ild(k + 1, (p + 1) % 3)
        return carry

    lax.fori_loop(0, (NCHW - 5) // 3, triple, 0)
    k = jnp.int32(NCHW - 2)
    ws(0)
    wi_g(k, 0)
    finish(k - 2, 1)
    ild(k + 1, 1)
    ws(1)
    wi_g(k + 1, 1)
    finish(k - 1, 2)
    finish(k, 0)
    finish(k + 1, 1)
    ws(2)
    ws(0)
    ws(1)


@functools.partial(
    pl.kernel,
    out_type=jax.ShapeDtypeStruct((NC, NPAD), jnp.float32),
    mesh=_mesh(),
    compiler_params=pltpu.CompilerParams(needs_layout_passes=False),
    scratch_types=[
        pltpu.VMEM((EPW,), jnp.int32),
        pltpu.VMEM((C,), jnp.int32),
        pltpu.VMEM((C,), jnp.float32),
        pltpu.VMEM_SHARED((NPAD,), jnp.float32),
    ],
)
def _deg_kernel(colall_hbm, zeros_hbm, degp_hbm, col_v, cidx, ones_v, deg_sh):
    c = lax.axis_index("c")
    s = lax.axis_index("s")
    wid = s * NC + c

    @pl.when(s == 0)
    def _zero():
        pltpu.sync_copy(zeros_hbm, deg_sh)

    pltpu.sync_copy(colall_hbm.at[pl.ds(wid * EPW, EPW)], col_v)

    def fill(i, carry):
        ones_v[pl.ds(i * L, L)] = jnp.full((L,), 1.0, jnp.float32)
        return carry

    lax.fori_loop(0, C // L, fill, 0)
    plsc.subcore_barrier()

    def chunk(k, carry):
        for g in range(C // L):
            cidx[pl.ds(g * L, L)] = col_v[pl.ds(k * C + g * L, L)]
        pltpu.sync_copy(ones_v, deg_sh.at[cidx], add=True)
        return carry

    lax.fori_loop(0, NCHUNK, chunk, 0)
    plsc.subcore_barrier()

    @pl.when(s == 0)
    def _out():
        pltpu.sync_copy(deg_sh, degp_hbm.at[c])


@functools.partial(
    pl.kernel,
    out_type=jax.ShapeDtypeStruct((NC, NPAD, D), jnp.float32),
    mesh=_mesh(),
    compiler_params=pltpu.CompilerParams(needs_layout_passes=False),
    scratch_types=[
        pltpu.VMEM((NPAD,), jnp.float32),
        pltpu.VMEM((PIECE,), jnp.float32),
        [pltpu.VMEM((C, D), jnp.float32) for _ in range(NBUF)],
        [pltpu.VMEM((3, C), jnp.int32) for _ in range(NBUF)],
        [pltpu.VMEM((C,), jnp.int32) for _ in range(NBUF)],
        pltpu.VMEM_SHARED((NPAD, D), jnp.float32),
        [pltpu.SemaphoreType.DMA for _ in range(NBUF)],
        [pltpu.SemaphoreType.DMA for _ in range(NBUF)],
        [pltpu.SemaphoreType.DMA for _ in range(NBUF)],
    ],
)
def _layer_kernel(packed_hbm, x_hbm, degp_hbm, zrow_hbm, part_hbm,
                  dis_v, piece, rows3, pbuf3, scidx3,
                  out_sh, gsem3, isem3, ssem3):
    c = lax.axis_index("c")
    s = lax.axis_index("s")
    wid = s * NC + c

    pltpu.sync_copy(zrow_hbm, out_sh.at[pl.ds(s * SLAB, SLAB)])

    pltpu.sync_copy(degp_hbm.at[0], dis_v)
    for pc in range(NPAD // PIECE):
        pltpu.sync_copy(degp_hbm.at[1, pl.ds(pc * PIECE, PIECE)], piece)

        def mkdis(i, carry, pc=pc):
            off = pc * PIECE + i * L
            dv = dis_v[pl.ds(off, L)] + piece[pl.ds(i * L, L)]
            dis_v[pl.ds(off, L)] = _rsqrt16(dv)
            return carry

        lax.fori_loop(0, PIECE // L, mkdis, 0)

    plsc.subcore_barrier()
    _propagate_pipe(wid, packed_hbm, x_hbm, out_sh, dis_v,
                    rows3, pbuf3, scidx3, gsem3, isem3, ssem3)
    plsc.subcore_barrier()
    pltpu.sync_copy(out_sh.at[pl.ds(s * SLAB, SLAB)],
                    part_hbm.at[c, pl.ds(s * SLAB, SLAB)])


TCB = 1024


def _tc_combine_body(a_ref, b_ref, o_ref):
    o_ref[...] = a_ref[...] + b_ref[...]


_tc_combine = pl.pallas_call(
    _tc_combine_body,
    grid=(NPAD // TCB,),
    in_specs=[pl.BlockSpec((TCB, D), lambda i: (i, 0))] * 2,
    out_specs=pl.BlockSpec((TCB, D), lambda i: (i, 0)),
    out_shape=jax.ShapeDtypeStruct((NPAD, D), jnp.float32),
)


def _tc_final_body(a_ref, b_ref, c_ref, d_ref, o_ref):
    o_ref[...] = (a_ref[...] + b_ref[...] + c_ref[...] + d_ref[...]) * (
        1.0 / 3.0)


_tc_final = pl.pallas_call(
    _tc_final_body,
    grid=(NPAD // TCB,),
    in_specs=[pl.BlockSpec((TCB, D), lambda i: (i, 0))] * 4,
    out_specs=pl.BlockSpec((TCB, D), lambda i: (i, 0)),
    out_shape=jax.ShapeDtypeStruct((NPAD, D), jnp.float32),
)


def kernel(edge_index, edge_weight, item_emb):
    x0 = jnp.zeros((NPAD, D), jnp.float32).at[:N].set(item_emb)
    zeros_deg = jnp.zeros((NPAD,), jnp.float32)
    zrow = jnp.zeros((SLAB, D), jnp.float32)
    rows_a = edge_index[0]
    cols_a = edge_index[1]
    ew_i = lax.bitcast_convert_type(edge_weight, jnp.int32)
    packed = jnp.stack([rows_a.reshape(E // C, C), cols_a.reshape(E // C, C),
                        ew_i.reshape(E // C, C)], axis=1)
    degp = _deg_kernel(cols_a, zeros_deg)
    part1 = _layer_kernel(packed, x0, degp, zrow)
    x1 = _tc_combine(part1[0], part1[1])
    part2 = _layer_kernel(packed, x1, degp, zrow)
    final = _tc_final(x0, x1, part2[0], part2[1])
    return final[:N]
